# baseline (speedup 1.0000x reference)
import functools

import jax
import jax.numpy as jnp
from jax import lax
from jax.experimental import pallas as pl
from jax.experimental.pallas import tpu as pltpu
from jax.experimental.pallas import tpu_sc as plsc

_CROP = 14
_NPOS = _CROP * _CROP
_NCHUNK = 13
_NPAD = _NCHUNK * 16
_PATCH = 16
_PPIX = _PATCH * _PATCH
_NWORK = 32


def _tec_body(table, scb, out, sciv, patch, outv, sem, osem):
    c_ax = lax.axis_index("c")
    s_ax = lax.axis_index("s")
    wid = s_ax * 2 + c_ax
    base = wid * 31 + jnp.minimum(wid, 8)
    nboxes = jnp.where(wid < 8, 32, 31)
    lanes = lax.iota(jnp.int32, 16)

    def box_body(i, carry):
        m = base + i
        pltpu.sync_copy(scb.at[m], sciv)
        cp0 = pltpu.async_copy(
            table.at[sciv.at[pl.ds(8 * _NPAD, 128)]], patch.at[pl.ds(0, 128)], sem)
        cp1 = pltpu.async_copy(
            table.at[sciv.at[pl.ds(8 * _NPAD + 128, 128)]], patch.at[pl.ds(128, 128)], sem)
        cp0.wait()
        cp1.wait()

        @pl.when(i > 0)
        def _drain_prev():
            pltpu.make_async_copy(outv, out.at[m], osem).wait()

        @plsc.parallel_loop(0, _NPOS, unroll=2)
        def _p_loop(p):
            sp = jnp.full((16,), p, jnp.int32)
            g0 = plsc.load_gather(sciv, [sp + 0 * _NPAD])
            g1 = plsc.load_gather(sciv, [sp + 1 * _NPAD])
            g2 = plsc.load_gather(sciv, [sp + 2 * _NPAD])
            g3 = plsc.load_gather(sciv, [sp + 3 * _NPAD])
            w0 = plsc.bitcast(plsc.load_gather(sciv, [sp + 4 * _NPAD]), jnp.float32)
            w1 = plsc.bitcast(plsc.load_gather(sciv, [sp + 5 * _NPAD]), jnp.float32)
            w2 = plsc.bitcast(plsc.load_gather(sciv, [sp + 6 * _NPAD]), jnp.float32)
            w3 = plsc.bitcast(plsc.load_gather(sciv, [sp + 7 * _NPAD]), jnp.float32)
            for c16 in range(16):
                cvec = lanes + c16 * 16
                v0 = plsc.load_gather(patch, [g0, cvec])
                v1 = plsc.load_gather(patch, [g1, cvec])
                v2 = plsc.load_gather(patch, [g2, cvec])
                v3 = plsc.load_gather(patch, [g3, cvec])
                acc = w0 * v0 + w1 * v1 + w2 * v2 + w3 * v3
                plsc.store_scatter(outv, [cvec * _NPOS + sp], acc)

        pltpu.async_copy(outv, out.at[m], osem)
        return carry

    lax.fori_loop(0, nboxes, box_body, 0)
    pltpu.make_async_copy(outv, out.at[base + nboxes - 1], osem).wait()


def _tr_body(in_ref, out_ref):
    x = in_ref[0]
    x = x.reshape(x.shape[0], -1)
    out_ref[...] = x.T


@jax.jit
def _nhwc_table(fm):
    n, c, h, w = fm.shape
    nq = h // 8
    return pl.pallas_call(
        _tr_body,
        grid=(n, nq),
        in_specs=[pl.BlockSpec((1, c, 8, w), lambda b, q: (b, 0, q, 0))],
        out_specs=pl.BlockSpec((8 * w, c), lambda b, q: (b * nq + q, 0)),
        out_shape=jax.ShapeDtypeStruct((n * h * w, c), jnp.float32),
    )(fm)


@jax.jit
def _roialign_sc(table, scb):
    nbox = scb.shape[0]
    nch = table.shape[1]
    mesh = plsc.VectorSubcoreMesh(core_axis_name="c", subcore_axis_name="s")
    return pl.kernel(
        _tec_body,
        out_type=jax.ShapeDtypeStruct((nbox, nch * _NPOS), jnp.float32),
        mesh=mesh,
        compiler_params=pltpu.CompilerParams(needs_layout_passes=False),
        scratch_types=[
            pltpu.VMEM((8 * _NPAD + 256,), jnp.int32),
            pltpu.VMEM((_PPIX, 256), jnp.float32),
            pltpu.VMEM((nch * _NPOS,), jnp.float32),
            pltpu.SemaphoreType.DMA,
            pltpu.SemaphoreType.DMA,
        ],
    )(table, scb)


def kernel(featuremap, boxes, box_ind):
    n, c, h, w = featuremap.shape
    nbox = boxes.shape[0]
    table = _nhwc_table(featuremap)

    x1 = boxes[:, 0:1]
    y1 = boxes[:, 1:2]
    x2 = boxes[:, 2:3]
    y2 = boxes[:, 3:4]
    spacing_w = (x2 - x1) / float(_CROP)
    spacing_h = (y2 - y1) / float(_CROP)
    nx0 = (x1 + spacing_w / 2 - 0.5) / float(w - 1)
    ny0 = (y1 + spacing_h / 2 - 0.5) / float(h - 1)
    nw_ = spacing_w * float(_CROP - 1) / float(w - 1)
    nh_ = spacing_h * float(_CROP - 1) / float(h - 1)
    y1n = ny0[:, 0]
    x1n = nx0[:, 0]
    y2n = (ny0 + nh_)[:, 0]
    x2n = (nx0 + nw_)[:, 0]

    ii = jnp.arange(_CROP, dtype=jnp.float32)
    in_y = y1n[:, None] * (h - 1) + ii[None, :] * (
        (y2n - y1n)[:, None] * (h - 1) / float(_CROP - 1))
    in_x = x1n[:, None] * (w - 1) + ii[None, :] * (
        (x2n - x1n)[:, None] * (w - 1) / float(_CROP - 1))
    valid_y = (in_y >= 0.0) & (in_y <= float(h - 1))
    valid_x = (in_x >= 0.0) & (in_x <= float(w - 1))
    y_lo_f = jnp.floor(in_y)
    x_lo_f = jnp.floor(in_x)
    y_lerp = in_y - y_lo_f
    x_lerp = in_x - x_lo_f
    y_lo = jnp.clip(y_lo_f, 0, h - 1).astype(jnp.int32)
    y_hi = jnp.clip(jnp.ceil(in_y), 0, h - 1).astype(jnp.int32)
    x_lo = jnp.clip(x_lo_f, 0, w - 1).astype(jnp.int32)
    x_hi = jnp.clip(jnp.ceil(in_x), 0, w - 1).astype(jnp.int32)

    y0 = jnp.clip(y_lo[:, 0], 0, h - _PATCH)
    x0 = jnp.clip(x_lo[:, 0], 0, w - _PATCH)
    ly_lo = jnp.clip(y_lo - y0[:, None], 0, _PATCH - 1)
    ly_hi = jnp.clip(y_hi - y0[:, None], 0, _PATCH - 1)
    lx_lo = jnp.clip(x_lo - x0[:, None], 0, _PATCH - 1)
    lx_hi = jnp.clip(x_hi - x0[:, None], 0, _PATCH - 1)

    pp = jnp.arange(_NPAD)
    piy = jnp.minimum(pp // _CROP, _CROP - 1)
    pix = pp % _CROP
    wy = y_lerp[:, piy]
    wx = x_lerp[:, pix]
    vmask = valid_y[:, piy] & valid_x[:, pix] & (pp < _NPOS)[None, :]
    vf = vmask.astype(jnp.float32)
    w_tl = (1.0 - wy) * (1.0 - wx) * vf
    w_tr = (1.0 - wy) * wx * vf
    w_bl = wy * (1.0 - wx) * vf
    w_br = wy * wx * vf
    p_tl = ly_lo[:, piy] * _PATCH + lx_lo[:, pix]
    p_tr = ly_lo[:, piy] * _PATCH + lx_hi[:, pix]
    p_bl = ly_hi[:, piy] * _PATCH + lx_lo[:, pix]
    p_br = ly_hi[:, piy] * _PATCH + lx_hi[:, pix]
    wts = jnp.concatenate([w_tl, w_tr, w_bl, w_br], axis=1)
    cidx = jnp.concatenate([p_tl, p_tr, p_bl, p_br], axis=1).astype(jnp.int32)

    dy = jnp.arange(_PATCH, dtype=jnp.int32)
    gidx = (box_ind[:, None, None] * (h * w)
            + (y0[:, None, None] + dy[None, :, None]) * w
            + (x0[:, None, None] + dy[None, None, :]))
    gidx = gidx.reshape(nbox, _PPIX).astype(jnp.int32)

    scb = jnp.concatenate(
        [cidx, jax.lax.bitcast_convert_type(wts, jnp.int32), gidx], axis=1)

    out = _roialign_sc(table, scb)
    return out.reshape(nbox, c, _CROP, _CROP)


# speedup vs baseline: 1.0596x; 1.0596x over previous
import functools

import jax
import jax.numpy as jnp
from jax import lax
from jax.experimental import pallas as pl
from jax.experimental.pallas import tpu as pltpu
from jax.experimental.pallas import tpu_sc as plsc

_CROP = 14
_NPOS = _CROP * _CROP
_NCHUNK = 13
_NPAD = _NCHUNK * 16
_PATCH = 16
_PPIX = _PATCH * _PATCH
_NWORK = 32


def _tec_body(table, scb, out, sciv, patch, outv, sem, osem):
    c_ax = lax.axis_index("c")
    s_ax = lax.axis_index("s")
    wid = s_ax * 2 + c_ax
    base = wid * 31 + jnp.minimum(wid, 8)
    nboxes = jnp.where(wid < 8, 32, 31)
    lanes = lax.iota(jnp.int32, 16)

    def box_body(i, carry):
        m = base + i
        pltpu.sync_copy(scb.at[m], sciv)
        cp0 = pltpu.async_copy(
            table.at[sciv.at[pl.ds(8 * _NPAD, 128)]], patch.at[pl.ds(0, 128)], sem)
        cp1 = pltpu.async_copy(
            table.at[sciv.at[pl.ds(8 * _NPAD + 128, 128)]], patch.at[pl.ds(128, 128)], sem)
        cp0.wait()
        cp1.wait()

        @pl.when(i > 0)
        def _drain_prev():
            pltpu.make_async_copy(outv, out.at[m], osem).wait()

        @plsc.parallel_loop(0, _NPOS, unroll=2)
        def _p_loop(p):
            sp = jnp.full((16,), p, jnp.int32)
            g0 = plsc.load_gather(sciv, [sp + 0 * _NPAD])
            g1 = plsc.load_gather(sciv, [sp + 1 * _NPAD])
            g2 = plsc.load_gather(sciv, [sp + 2 * _NPAD])
            g3 = plsc.load_gather(sciv, [sp + 3 * _NPAD])
            w0 = plsc.bitcast(plsc.load_gather(sciv, [sp + 4 * _NPAD]), jnp.float32)
            w1 = plsc.bitcast(plsc.load_gather(sciv, [sp + 5 * _NPAD]), jnp.float32)
            w2 = plsc.bitcast(plsc.load_gather(sciv, [sp + 6 * _NPAD]), jnp.float32)
            w3 = plsc.bitcast(plsc.load_gather(sciv, [sp + 7 * _NPAD]), jnp.float32)
            for c16 in range(16):
                cvec = lanes + c16 * 16
                v0 = plsc.load_gather(patch, [g0, cvec])
                v1 = plsc.load_gather(patch, [g1, cvec])
                v2 = plsc.load_gather(patch, [g2, cvec])
                v3 = plsc.load_gather(patch, [g3, cvec])
                acc = w0 * v0 + w1 * v1 + w2 * v2 + w3 * v3
                plsc.store_scatter(outv, [cvec * _NPOS + sp], acc)

        pltpu.async_copy(outv, out.at[m], osem)
        return carry

    lax.fori_loop(0, nboxes, box_body, 0)
    pltpu.make_async_copy(outv, out.at[base + nboxes - 1], osem).wait()


@jax.jit
def _roialign_sc(table, scb):
    nbox = scb.shape[0]
    nch = table.shape[1]
    mesh = plsc.VectorSubcoreMesh(core_axis_name="c", subcore_axis_name="s")
    return pl.kernel(
        _tec_body,
        out_type=jax.ShapeDtypeStruct((nbox, nch * _NPOS), jnp.float32),
        mesh=mesh,
        compiler_params=pltpu.CompilerParams(needs_layout_passes=False),
        scratch_types=[
            pltpu.VMEM((8 * _NPAD + 256,), jnp.int32),
            pltpu.VMEM((_PPIX, 256), jnp.float32),
            pltpu.VMEM((nch * _NPOS,), jnp.float32),
            pltpu.SemaphoreType.DMA,
            pltpu.SemaphoreType.DMA,
        ],
    )(table, scb)


def kernel(featuremap, boxes, box_ind):
    n, c, h, w = featuremap.shape
    nbox = boxes.shape[0]
    table = jnp.transpose(featuremap, (0, 2, 3, 1)).reshape(n * h * w, c)

    x1 = boxes[:, 0:1]
    y1 = boxes[:, 1:2]
    x2 = boxes[:, 2:3]
    y2 = boxes[:, 3:4]
    spacing_w = (x2 - x1) / float(_CROP)
    spacing_h = (y2 - y1) / float(_CROP)
    nx0 = (x1 + spacing_w / 2 - 0.5) / float(w - 1)
    ny0 = (y1 + spacing_h / 2 - 0.5) / float(h - 1)
    nw_ = spacing_w * float(_CROP - 1) / float(w - 1)
    nh_ = spacing_h * float(_CROP - 1) / float(h - 1)
    y1n = ny0[:, 0]
    x1n = nx0[:, 0]
    y2n = (ny0 + nh_)[:, 0]
    x2n = (nx0 + nw_)[:, 0]

    ii = jnp.arange(_CROP, dtype=jnp.float32)
    in_y = y1n[:, None] * (h - 1) + ii[None, :] * (
        (y2n - y1n)[:, None] * (h - 1) / float(_CROP - 1))
    in_x = x1n[:, None] * (w - 1) + ii[None, :] * (
        (x2n - x1n)[:, None] * (w - 1) / float(_CROP - 1))
    valid_y = (in_y >= 0.0) & (in_y <= float(h - 1))
    valid_x = (in_x >= 0.0) & (in_x <= float(w - 1))
    y_lo_f = jnp.floor(in_y)
    x_lo_f = jnp.floor(in_x)
    y_lerp = in_y - y_lo_f
    x_lerp = in_x - x_lo_f
    y_lo = jnp.clip(y_lo_f, 0, h - 1).astype(jnp.int32)
    y_hi = jnp.clip(jnp.ceil(in_y), 0, h - 1).astype(jnp.int32)
    x_lo = jnp.clip(x_lo_f, 0, w - 1).astype(jnp.int32)
    x_hi = jnp.clip(jnp.ceil(in_x), 0, w - 1).astype(jnp.int32)

    y0 = jnp.clip(y_lo[:, 0], 0, h - _PATCH)
    x0 = jnp.clip(x_lo[:, 0], 0, w - _PATCH)
    ly_lo = jnp.clip(y_lo - y0[:, None], 0, _PATCH - 1)
    ly_hi = jnp.clip(y_hi - y0[:, None], 0, _PATCH - 1)
    lx_lo = jnp.clip(x_lo - x0[:, None], 0, _PATCH - 1)
    lx_hi = jnp.clip(x_hi - x0[:, None], 0, _PATCH - 1)

    pp = jnp.arange(_NPAD)
    piy = jnp.minimum(pp // _CROP, _CROP - 1)
    pix = pp % _CROP
    wy = y_lerp[:, piy]
    wx = x_lerp[:, pix]
    vmask = valid_y[:, piy] & valid_x[:, pix] & (pp < _NPOS)[None, :]
    vf = vmask.astype(jnp.float32)
    w_tl = (1.0 - wy) * (1.0 - wx) * vf
    w_tr = (1.0 - wy) * wx * vf
    w_bl = wy * (1.0 - wx) * vf
    w_br = wy * wx * vf
    p_tl = ly_lo[:, piy] * _PATCH + lx_lo[:, pix]
    p_tr = ly_lo[:, piy] * _PATCH + lx_hi[:, pix]
    p_bl = ly_hi[:, piy] * _PATCH + lx_lo[:, pix]
    p_br = ly_hi[:, piy] * _PATCH + lx_hi[:, pix]
    wts = jnp.concatenate([w_tl, w_tr, w_bl, w_br], axis=1)
    cidx = jnp.concatenate([p_tl, p_tr, p_bl, p_br], axis=1).astype(jnp.int32)

    dy = jnp.arange(_PATCH, dtype=jnp.int32)
    gidx = (box_ind[:, None, None] * (h * w)
            + (y0[:, None, None] + dy[None, :, None]) * w
            + (x0[:, None, None] + dy[None, None, :]))
    gidx = gidx.reshape(nbox, _PPIX).astype(jnp.int32)

    scb = jnp.concatenate(
        [cidx, jax.lax.bitcast_convert_type(wts, jnp.int32), gidx], axis=1)

    out = _roialign_sc(table, scb)
    return out.reshape(nbox, c, _CROP, _CROP)


# speedup vs baseline: 1.0602x; 1.0006x over previous
import functools

import jax
import jax.numpy as jnp
from jax import lax
from jax.experimental import pallas as pl
from jax.experimental.pallas import tpu as pltpu
from jax.experimental.pallas import tpu_sc as plsc

_CROP = 14
_NPOS = _CROP * _CROP
_NCHUNK = 13
_NPAD = _NCHUNK * 16
_PATCH = 16
_PPIX = _PATCH * _PATCH
_NWORK = 32


def _tec_body(table, scb, out, sciv, patch, outv, sem, osem):
    c_ax = lax.axis_index("c")
    s_ax = lax.axis_index("s")
    wid = s_ax * 2 + c_ax
    base = wid * 31 + jnp.minimum(wid, 8)
    nboxes = jnp.where(wid < 8, 32, 31)
    lanes = lax.iota(jnp.int32, 16)

    def box_body(i, carry):
        m = base + i
        pltpu.sync_copy(scb.at[m], sciv)
        cp0 = pltpu.async_copy(
            table.at[sciv.at[pl.ds(8 * _NPAD, 128)]], patch.at[pl.ds(0, 128)], sem)
        cp1 = pltpu.async_copy(
            table.at[sciv.at[pl.ds(8 * _NPAD + 128, 128)]], patch.at[pl.ds(128, 128)], sem)
        cp0.wait()
        cp1.wait()

        @pl.when(i > 0)
        def _drain_prev():
            pltpu.make_async_copy(outv, out.at[m], osem).wait()

        @plsc.parallel_loop(0, _NPOS, unroll=2)
        def _p_loop(p):
            sp = jnp.full((16,), p, jnp.int32)
            g0 = plsc.load_gather(sciv, [sp + 0 * _NPAD])
            g1 = plsc.load_gather(sciv, [sp + 1 * _NPAD])
            g2 = plsc.load_gather(sciv, [sp + 2 * _NPAD])
            g3 = plsc.load_gather(sciv, [sp + 3 * _NPAD])
            w0 = plsc.bitcast(plsc.load_gather(sciv, [sp + 4 * _NPAD]), jnp.float32)
            w1 = plsc.bitcast(plsc.load_gather(sciv, [sp + 5 * _NPAD]), jnp.float32)
            w2 = plsc.bitcast(plsc.load_gather(sciv, [sp + 6 * _NPAD]), jnp.float32)
            w3 = plsc.bitcast(plsc.load_gather(sciv, [sp + 7 * _NPAD]), jnp.float32)
            ovec = lanes * _NPOS + sp
            for c16 in range(16):
                cvec = lanes + c16 * 16
                v0 = plsc.load_gather(patch, [g0, cvec])
                v1 = plsc.load_gather(patch, [g1, cvec])
                v2 = plsc.load_gather(patch, [g2, cvec])
                v3 = plsc.load_gather(patch, [g3, cvec])
                acc = w0 * v0 + w1 * v1 + w2 * v2 + w3 * v3
                plsc.store_scatter(outv, [ovec + c16 * (16 * _NPOS)], acc)

        pltpu.async_copy(outv, out.at[m], osem)
        return carry

    lax.fori_loop(0, nboxes, box_body, 0)
    pltpu.make_async_copy(outv, out.at[base + nboxes - 1], osem).wait()


@jax.jit
def _roialign_sc(table, scb):
    nbox = scb.shape[0]
    nch = table.shape[1]
    mesh = plsc.VectorSubcoreMesh(core_axis_name="c", subcore_axis_name="s")
    return pl.kernel(
        _tec_body,
        out_type=jax.ShapeDtypeStruct((nbox, nch * _NPOS), jnp.float32),
        mesh=mesh,
        compiler_params=pltpu.CompilerParams(needs_layout_passes=False),
        scratch_types=[
            pltpu.VMEM((8 * _NPAD + 256,), jnp.int32),
            pltpu.VMEM((_PPIX, 256), jnp.float32),
            pltpu.VMEM((nch * _NPOS,), jnp.float32),
            pltpu.SemaphoreType.DMA,
            pltpu.SemaphoreType.DMA,
        ],
    )(table, scb)


def kernel(featuremap, boxes, box_ind):
    n, c, h, w = featuremap.shape
    nbox = boxes.shape[0]
    table = jnp.transpose(featuremap, (0, 2, 3, 1)).reshape(n * h * w, c)

    x1 = boxes[:, 0:1]
    y1 = boxes[:, 1:2]
    x2 = boxes[:, 2:3]
    y2 = boxes[:, 3:4]
    spacing_w = (x2 - x1) / float(_CROP)
    spacing_h = (y2 - y1) / float(_CROP)
    nx0 = (x1 + spacing_w / 2 - 0.5) / float(w - 1)
    ny0 = (y1 + spacing_h / 2 - 0.5) / float(h - 1)
    nw_ = spacing_w * float(_CROP - 1) / float(w - 1)
    nh_ = spacing_h * float(_CROP - 1) / float(h - 1)
    y1n = ny0[:, 0]
    x1n = nx0[:, 0]
    y2n = (ny0 + nh_)[:, 0]
    x2n = (nx0 + nw_)[:, 0]

    ii = jnp.arange(_CROP, dtype=jnp.float32)
    in_y = y1n[:, None] * (h - 1) + ii[None, :] * (
        (y2n - y1n)[:, None] * (h - 1) / float(_CROP - 1))
    in_x = x1n[:, None] * (w - 1) + ii[None, :] * (
        (x2n - x1n)[:, None] * (w - 1) / float(_CROP - 1))
    valid_y = (in_y >= 0.0) & (in_y <= float(h - 1))
    valid_x = (in_x >= 0.0) & (in_x <= float(w - 1))
    y_lo_f = jnp.floor(in_y)
    x_lo_f = jnp.floor(in_x)
    y_lerp = in_y - y_lo_f
    x_lerp = in_x - x_lo_f
    y_lo = jnp.clip(y_lo_f, 0, h - 1).astype(jnp.int32)
    y_hi = jnp.clip(jnp.ceil(in_y), 0, h - 1).astype(jnp.int32)
    x_lo = jnp.clip(x_lo_f, 0, w - 1).astype(jnp.int32)
    x_hi = jnp.clip(jnp.ceil(in_x), 0, w - 1).astype(jnp.int32)

    y0 = jnp.clip(y_lo[:, 0], 0, h - _PATCH)
    x0 = jnp.clip(x_lo[:, 0], 0, w - _PATCH)
    ly_lo = jnp.clip(y_lo - y0[:, None], 0, _PATCH - 1)
    ly_hi = jnp.clip(y_hi - y0[:, None], 0, _PATCH - 1)
    lx_lo = jnp.clip(x_lo - x0[:, None], 0, _PATCH - 1)
    lx_hi = jnp.clip(x_hi - x0[:, None], 0, _PATCH - 1)

    pp = jnp.arange(_NPAD)
    piy = jnp.minimum(pp // _CROP, _CROP - 1)
    pix = pp % _CROP
    wy = y_lerp[:, piy]
    wx = x_lerp[:, pix]
    vmask = valid_y[:, piy] & valid_x[:, pix] & (pp < _NPOS)[None, :]
    vf = vmask.astype(jnp.float32)
    w_tl = (1.0 - wy) * (1.0 - wx) * vf
    w_tr = (1.0 - wy) * wx * vf
    w_bl = wy * (1.0 - wx) * vf
    w_br = wy * wx * vf
    p_tl = ly_lo[:, piy] * _PATCH + lx_lo[:, pix]
    p_tr = ly_lo[:, piy] * _PATCH + lx_hi[:, pix]
    p_bl = ly_hi[:, piy] * _PATCH + lx_lo[:, pix]
    p_br = ly_hi[:, piy] * _PATCH + lx_hi[:, pix]
    wts = jnp.concatenate([w_tl, w_tr, w_bl, w_br], axis=1)
    cidx = jnp.concatenate([p_tl, p_tr, p_bl, p_br], axis=1).astype(jnp.int32)

    dy = jnp.arange(_PATCH, dtype=jnp.int32)
    gidx = (box_ind[:, None, None] * (h * w)
            + (y0[:, None, None] + dy[None, :, None]) * w
            + (x0[:, None, None] + dy[None, None, :]))
    gidx = gidx.reshape(nbox, _PPIX).astype(jnp.int32)

    scb = jnp.concatenate(
        [cidx, jax.lax.bitcast_convert_type(wts, jnp.int32), gidx], axis=1)

    out = _roialign_sc(table, scb)
    return out.reshape(nbox, c, _CROP, _CROP)
